# baseline (speedup 1.0000x reference)
import functools

import jax
import jax.numpy as jnp
from jax import lax
from jax.experimental import pallas as pl
from jax.experimental.pallas import tpu as pltpu
from jax.experimental.pallas import tpu_sc as plsc

N_TOKENS = 4096
D_IN = 1024
D_OUT = 1024
N_EXPERTS = 8
H = 1024

T = 128
P = N_TOKENS + N_EXPERTS * T
NT = P // T


_GC = 512
_GN = N_TOKENS // _GC


def _gating_body(x_ref, w1_ref, b1_ref, w2_ref, b2_ref, w3_ref, b3_ref,
                 logits_ref, ids_ref, dest_ref, ci_ref, lg_scr):
    i = pl.program_id(0)

    @pl.when(i < _GN)
    def _():
        h = jnp.maximum(jnp.dot(x_ref[...], w1_ref[...],
                                preferred_element_type=jnp.float32)
                        + b1_ref[...], 0.0)
        h = jnp.maximum(jnp.dot(h, w2_ref[...],
                                preferred_element_type=jnp.float32)
                        + b2_ref[...], 0.0)
        lg = jnp.dot(h, w3_ref[...],
                     preferred_element_type=jnp.float32) + b3_ref[...]
        logits_ref[...] = lg
        lg_scr[pl.ds(i * _GC, _GC), :] = lg

    @pl.when(i == _GN)
    def _():
        lg = lg_scr[...]
        m = jnp.max(lg, axis=1, keepdims=True)
        cols = lax.broadcasted_iota(jnp.int32, lg.shape, 1)
        ids = jnp.min(jnp.where(lg == m, cols, N_EXPERTS), axis=1,
                      keepdims=True)
        ids_ref[...] = ids

        oh = (cols == ids).astype(jnp.int32)
        cs = oh
        sh = 1
        while sh < N_TOKENS:
            cs = cs + jnp.concatenate(
                [jnp.zeros((sh, N_EXPERTS), jnp.int32), cs[:-sh, :]], axis=0)
            sh *= 2
        counts = cs[N_TOKENS - 1:N_TOKENS, :]
        tile_cnt = (counts + (T - 1)) // T
        ci = tile_cnt
        sh = 1
        while sh < N_EXPERTS:
            ci = ci + jnp.concatenate(
                [jnp.zeros((1, sh), jnp.int32), ci[:, :-sh]], axis=1)
            sh *= 2
        ci_ref[...] = ci
        pad_off = (ci - tile_cnt) * T
        rank = jnp.sum(oh * (cs - 1), axis=1, keepdims=True)
        base = jnp.sum(oh * pad_off, axis=1, keepdims=True)
        dest_ref[...] = base + rank


def _gating(x, gW1, gb1, gW2, gb2, gW3, gb3):
    last = _GN - 1
    logits, ids, dest, ci = pl.pallas_call(
        _gating_body,
        grid=(_GN + 1,),
        in_specs=[
            pl.BlockSpec((_GC, D_IN), lambda i: (jnp.minimum(i, last), 0)),
            pl.BlockSpec((D_IN, 64), lambda i: (0, 0)),
            pl.BlockSpec((1, 64), lambda i: (0, 0)),
            pl.BlockSpec((64, 32), lambda i: (0, 0)),
            pl.BlockSpec((1, 32), lambda i: (0, 0)),
            pl.BlockSpec((32, N_EXPERTS), lambda i: (0, 0)),
            pl.BlockSpec((1, N_EXPERTS), lambda i: (0, 0)),
        ],
        out_specs=(
            pl.BlockSpec((_GC, N_EXPERTS), lambda i: (jnp.minimum(i, last), 0)),
            pl.BlockSpec((N_TOKENS, 1), lambda i: (0, 0)),
            pl.BlockSpec((N_TOKENS, 1), lambda i: (0, 0)),
            pl.BlockSpec((1, N_EXPERTS), lambda i: (0, 0)),
        ),
        out_shape=(
            jax.ShapeDtypeStruct((N_TOKENS, N_EXPERTS), jnp.float32),
            jax.ShapeDtypeStruct((N_TOKENS, 1), jnp.int32),
            jax.ShapeDtypeStruct((N_TOKENS, 1), jnp.int32),
            jax.ShapeDtypeStruct((1, N_EXPERTS), jnp.int32),
        ),
        scratch_shapes=[pltpu.VMEM((N_TOKENS, N_EXPERTS), jnp.float32)],
        compiler_params=pltpu.CompilerParams(
            dimension_semantics=("arbitrary",)),
    )(x, gW1, gb1.reshape(1, -1), gW2, gb2.reshape(1, -1), gW3,
      gb3.reshape(1, -1))
    return logits, ids.reshape(N_TOKENS), dest.reshape(N_TOKENS), ci.reshape(N_EXPERTS)


@functools.lru_cache(maxsize=None)
def _make_row_mover(n_rows, n_rows_store, d, tag, scatter):
    info = plsc.get_sparse_core_info()
    nc, ns = info.num_cores, info.num_subcores
    nw = nc * ns
    per_w = n_rows // nw
    ch = 32
    n_ch = per_w // ch
    mesh = plsc.VectorSubcoreMesh(core_axis_name="c", subcore_axis_name="s")

    @functools.partial(
        pl.kernel,
        mesh=mesh,
        name=tag,
        out_type=jax.ShapeDtypeStruct((n_rows_store, d), jnp.float32),
        scratch_types=[
            pltpu.VMEM((n_ch, ch), jnp.int32),
            pltpu.VMEM((ch, d), jnp.float32),
            pltpu.VMEM((ch, d), jnp.float32),
            pltpu.SemaphoreType.DMA,
            pltpu.SemaphoreType.DMA,
            pltpu.SemaphoreType.DMA,
            pltpu.SemaphoreType.DMA,
        ],
    )
    def mover(table_hbm, idx_hbm, out_hbm, idx_v, buf0, buf1,
              gsem0, gsem1, osem0, osem1):
        wid = lax.axis_index("s") * nc + lax.axis_index("c")
        pltpu.sync_copy(idx_hbm.at[wid], idx_v)
        base = wid * per_w
        bufs, gsems, osems = (buf0, buf1), (gsem0, gsem1), (osem0, osem1)

        def start_in(c):
            if scatter:
                src = table_hbm.at[pl.ds(base + c * ch, ch)]
            else:
                src = table_hbm.at[idx_v.at[c]]
            return pltpu.async_copy(src, bufs[c % 2], gsems[c % 2])

        def start_out(c):
            if scatter:
                dst = out_hbm.at[idx_v.at[c]]
            else:
                dst = out_hbm.at[pl.ds(base + c * ch, ch)]
            return pltpu.async_copy(bufs[c % 2], dst, osems[c % 2])

        cp = start_in(0)
        outcp = [None, None]
        for c in range(n_ch):
            cp.wait()
            outcp[c % 2] = start_out(c)
            if c + 1 < n_ch:
                if outcp[(c + 1) % 2] is not None:
                    outcp[(c + 1) % 2].wait()
                cp = start_in(c + 1)
        outcp[(n_ch - 1) % 2].wait()
        if n_ch >= 2:
            outcp[n_ch % 2].wait()

    def run(table, idx):
        return mover(table, idx.reshape(nw, n_ch, ch))

    return run


def _expert_of(t, ci):
    e = jnp.int32(0)
    for k in range(N_EXPERTS):
        e = e + jnp.where(t >= ci[k], 1, 0).astype(jnp.int32)
    return jnp.minimum(e, N_EXPERTS - 1)


def _mlp_body(ci_ref, xs_ref, b1_ref, b2_ref, b3_ref,
              w1_hbm, w2_hbm, w3_hbm, o_ref,
              w1buf, w2buf, w3buf, sems):
    t = pl.program_id(0)

    def exp_of(tt):
        e = jnp.int32(0)
        for k in range(N_EXPERTS):
            e = e + jnp.where(tt >= ci_ref[k], 1, 0).astype(jnp.int32)
        return jnp.minimum(e, N_EXPERTS - 1)

    e_t = exp_of(t)
    total = ci_ref[N_EXPERTS - 1]
    new_run = jnp.logical_or(
        t == 0, jnp.logical_and(t < total, e_t != exp_of(t - 1)))
    run_idx = jnp.int32(0)
    prev = jnp.int32(0)
    for k in range(N_EXPERTS):
        present = ci_ref[k] > prev
        run_idx = run_idx + jnp.where(jnp.logical_and(k < e_t, present),
                                      1, 0).astype(jnp.int32)
        prev = ci_ref[k]
    p = lax.rem(run_idx, 3)
    nxt_tile = ci_ref[jnp.minimum(e_t, N_EXPERTS - 1)]
    has_next = nxt_tile < total
    e_next = exp_of(nxt_tile)
    nxt2_tile = ci_ref[jnp.minimum(e_next, N_EXPERTS - 1)]
    has_next2 = jnp.logical_and(has_next, nxt2_tile < total)
    e_next2 = exp_of(nxt2_tile)

    def issue(e_idx, parity):
        pltpu.make_async_copy(w1_hbm.at[e_idx], w1buf.at[parity],
                              sems.at[parity]).start()
        pltpu.make_async_copy(w2_hbm.at[e_idx], w2buf.at[parity],
                              sems.at[parity]).start()
        pltpu.make_async_copy(w3_hbm.at[e_idx], w3buf.at[parity],
                              sems.at[parity]).start()

    @pl.when(jnp.logical_and(new_run, t == 0))
    def _():
        issue(e_t, p)

        @pl.when(has_next)
        def _():
            issue(e_next, lax.rem(run_idx + 1, 3))

    @pl.when(new_run)
    def _():
        pltpu.make_async_copy(w1_hbm.at[0], w1buf.at[p], sems.at[p]).wait()
        pltpu.make_async_copy(w2_hbm.at[0], w2buf.at[p], sems.at[p]).wait()
        pltpu.make_async_copy(w3_hbm.at[0], w3buf.at[p], sems.at[p]).wait()

    @pl.when(jnp.logical_and(new_run, has_next2))
    def _():
        issue(e_next2, lax.rem(run_idx + 2, 3))

    @pl.when(t < total)
    def _():
        h = jnp.maximum(jnp.dot(xs_ref[...], w1buf[p],
                                preferred_element_type=jnp.float32)
                        + b1_ref[0], 0.0)
        h = jnp.maximum(jnp.dot(h, w2buf[p],
                                preferred_element_type=jnp.float32)
                        + b2_ref[0], 0.0)
        o_ref[...] = jnp.dot(h, w3buf[p],
                             preferred_element_type=jnp.float32) + b3_ref[0]


def _expert_mlp(ci, x_sorted, eW1, eb1, eW2, eb2, eW3, eb3):
    def wmap(t, ci_ref):
        return (_expert_of(t, ci_ref), 0, 0)

    grid_spec = pltpu.PrefetchScalarGridSpec(
        num_scalar_prefetch=1,
        grid=(NT,),
        in_specs=[
            pl.BlockSpec((T, D_IN), lambda t, ci_ref: (t, 0)),
            pl.BlockSpec((1, 1, H), wmap),
            pl.BlockSpec((1, 1, H), wmap),
            pl.BlockSpec((1, 1, D_OUT), wmap),
            pl.BlockSpec(memory_space=pl.ANY),
            pl.BlockSpec(memory_space=pl.ANY),
            pl.BlockSpec(memory_space=pl.ANY),
        ],
        out_specs=pl.BlockSpec((T, D_OUT), lambda t, ci_ref: (t, 0)),
        scratch_shapes=[
            pltpu.VMEM((3, D_IN, H), jnp.float32),
            pltpu.VMEM((3, H, H), jnp.float32),
            pltpu.VMEM((3, H, D_OUT), jnp.float32),
            pltpu.SemaphoreType.DMA((3,)),
        ],
    )
    return pl.pallas_call(
        _mlp_body,
        grid_spec=grid_spec,
        out_shape=jax.ShapeDtypeStruct((P, D_OUT), jnp.float32),
        compiler_params=pltpu.CompilerParams(
            dimension_semantics=("arbitrary",)),
    )(ci, x_sorted, eb1.reshape(N_EXPERTS, 1, H),
      eb2.reshape(N_EXPERTS, 1, H), eb3.reshape(N_EXPERTS, 1, D_OUT),
      eW1, eW2, eW3)


def kernel(x, gW1, gb1, gW2, gb2, gW3, gb3, eW1, eb1, eW2, eb2, eW3, eb3):
    logits, cluster_ids, dest, ci = _gating(x, gW1, gb1, gW2, gb2, gW3, gb3)
    x_sorted = _make_row_mover(N_TOKENS, P, D_IN, "dispatch_scatter", True)(x, dest)
    y_sorted = _expert_mlp(ci, x_sorted, eW1, eb1, eW2, eb2, eW3, eb3)
    outputs = _make_row_mover(N_TOKENS, N_TOKENS, D_OUT, "undispatch_gather",
                              False)(y_sorted, dest)
    return outputs, cluster_ids, logits


# speedup vs baseline: 1.0521x; 1.0521x over previous
import functools

import jax
import jax.numpy as jnp
from jax import lax
from jax.experimental import pallas as pl
from jax.experimental.pallas import tpu as pltpu
from jax.experimental.pallas import tpu_sc as plsc

N_TOKENS = 4096
D_IN = 1024
D_OUT = 1024
N_EXPERTS = 8
H = 1024

T = 128
P = N_TOKENS + N_EXPERTS * T
NT = P // T


def _dot_nt(a, bt):
    return lax.dot_general(a, bt, (((1,), (1,)), ((), ())),
                           preferred_element_type=jnp.float32)


def _gating_body(x_ref, w1t_ref, b1_ref, w2t_ref, b2_ref, w3t_ref, b3_ref,
                 logits_ref, ids_ref, dest_ref, ci_ref):
    h = jnp.maximum(_dot_nt(x_ref[...], w1t_ref[...]) + b1_ref[...], 0.0)
    h = jnp.maximum(_dot_nt(h, w2t_ref[...]) + b2_ref[...], 0.0)
    lg = _dot_nt(h, w3t_ref[...]) + b3_ref[...]
    logits_ref[...] = lg

    m = jnp.max(lg, axis=1, keepdims=True)
    cols = lax.broadcasted_iota(jnp.int32, lg.shape, 1)
    ids = jnp.min(jnp.where(lg == m, cols, N_EXPERTS), axis=1, keepdims=True)
    ids_ref[...] = ids

    oh = (cols == ids).astype(jnp.int32)
    cs = oh
    sh = 1
    while sh < N_TOKENS:
        cs = cs + jnp.concatenate(
            [jnp.zeros((sh, N_EXPERTS), jnp.int32), cs[:-sh, :]], axis=0)
        sh *= 2
    counts = cs[N_TOKENS - 1:N_TOKENS, :]
    tile_cnt = (counts + (T - 1)) // T
    ci = tile_cnt
    sh = 1
    while sh < N_EXPERTS:
        ci = ci + jnp.concatenate(
            [jnp.zeros((1, sh), jnp.int32), ci[:, :-sh]], axis=1)
        sh *= 2
    ci_ref[...] = ci
    pad_off = (ci - tile_cnt) * T
    rank = jnp.sum(oh * (cs - 1), axis=1, keepdims=True)
    base = jnp.sum(oh * pad_off, axis=1, keepdims=True)
    dest_ref[...] = base + rank


def _gating(x, gW1, gb1, gW2, gb2, gW3, gb3):
    logits, ids, dest, ci = pl.pallas_call(
        _gating_body,
        out_shape=(
            jax.ShapeDtypeStruct((N_TOKENS, N_EXPERTS), jnp.float32),
            jax.ShapeDtypeStruct((N_TOKENS, 1), jnp.int32),
            jax.ShapeDtypeStruct((N_TOKENS, 1), jnp.int32),
            jax.ShapeDtypeStruct((1, N_EXPERTS), jnp.int32),
        ),
    )(x, gW1.T, gb1.reshape(1, -1), gW2.T, gb2.reshape(1, -1), gW3.T,
      gb3.reshape(1, -1))
    return logits, ids.reshape(N_TOKENS), dest.reshape(N_TOKENS), ci.reshape(N_EXPERTS)


@functools.lru_cache(maxsize=None)
def _make_row_mover(n_rows, n_rows_store, d, tag, scatter):
    info = plsc.get_sparse_core_info()
    nc, ns = info.num_cores, info.num_subcores
    nw = nc * ns
    per_w = n_rows // nw
    ch = 32
    n_ch = per_w // ch
    mesh = plsc.VectorSubcoreMesh(core_axis_name="c", subcore_axis_name="s")

    @functools.partial(
        pl.kernel,
        mesh=mesh,
        name=tag,
        out_type=jax.ShapeDtypeStruct((n_rows_store, d), jnp.float32),
        scratch_types=[
            pltpu.VMEM((n_ch, ch), jnp.int32),
            pltpu.VMEM((ch, d), jnp.float32),
            pltpu.VMEM((ch, d), jnp.float32),
            pltpu.SemaphoreType.DMA,
            pltpu.SemaphoreType.DMA,
            pltpu.SemaphoreType.DMA,
            pltpu.SemaphoreType.DMA,
        ],
    )
    def mover(table_hbm, idx_hbm, out_hbm, idx_v, buf0, buf1,
              gsem0, gsem1, osem0, osem1):
        wid = lax.axis_index("s") * nc + lax.axis_index("c")
        pltpu.sync_copy(idx_hbm.at[wid], idx_v)
        base = wid * per_w
        bufs, gsems, osems = (buf0, buf1), (gsem0, gsem1), (osem0, osem1)

        def start_in(c):
            if scatter:
                src = table_hbm.at[pl.ds(base + c * ch, ch)]
            else:
                src = table_hbm.at[idx_v.at[c]]
            return pltpu.async_copy(src, bufs[c % 2], gsems[c % 2])

        def start_out(c):
            if scatter:
                dst = out_hbm.at[idx_v.at[c]]
            else:
                dst = out_hbm.at[pl.ds(base + c * ch, ch)]
            return pltpu.async_copy(bufs[c % 2], dst, osems[c % 2])

        cp = start_in(0)
        outcp = [None, None]
        for c in range(n_ch):
            cp.wait()
            outcp[c % 2] = start_out(c)
            if c + 1 < n_ch:
                if outcp[(c + 1) % 2] is not None:
                    outcp[(c + 1) % 2].wait()
                cp = start_in(c + 1)
        outcp[(n_ch - 1) % 2].wait()
        if n_ch >= 2:
            outcp[n_ch % 2].wait()

    def run(table, idx):
        return mover(table, idx.reshape(nw, n_ch, ch))

    return run


def _expert_of(t, ci):
    e = jnp.int32(0)
    for k in range(N_EXPERTS):
        e = e + jnp.where(t >= ci[k], 1, 0).astype(jnp.int32)
    return jnp.minimum(e, N_EXPERTS - 1)


def _mlp_body(ci_ref, xs_ref, b1_ref, b2_ref, b3_ref,
              w1_hbm, w2_hbm, w3_hbm, o_ref,
              w1buf, w2buf, w3buf, sems):
    t = pl.program_id(0)

    def exp_of(tt):
        e = jnp.int32(0)
        for k in range(N_EXPERTS):
            e = e + jnp.where(tt >= ci_ref[k], 1, 0).astype(jnp.int32)
        return jnp.minimum(e, N_EXPERTS - 1)

    e_t = exp_of(t)
    total = ci_ref[N_EXPERTS - 1]
    new_run = jnp.logical_or(
        t == 0, jnp.logical_and(t < total, e_t != exp_of(t - 1)))
    run_idx = jnp.int32(0)
    prev = jnp.int32(0)
    for k in range(N_EXPERTS):
        present = ci_ref[k] > prev
        run_idx = run_idx + jnp.where(jnp.logical_and(k < e_t, present),
                                      1, 0).astype(jnp.int32)
        prev = ci_ref[k]
    p = lax.rem(run_idx, 3)
    nxt_tile = ci_ref[jnp.minimum(e_t, N_EXPERTS - 1)]
    has_next = nxt_tile < total
    e_next = exp_of(nxt_tile)
    nxt2_tile = ci_ref[jnp.minimum(e_next, N_EXPERTS - 1)]
    has_next2 = jnp.logical_and(has_next, nxt2_tile < total)
    e_next2 = exp_of(nxt2_tile)

    def issue(e_idx, parity):
        pltpu.make_async_copy(w1_hbm.at[e_idx], w1buf.at[parity],
                              sems.at[parity]).start()
        pltpu.make_async_copy(w2_hbm.at[e_idx], w2buf.at[parity],
                              sems.at[parity]).start()
        pltpu.make_async_copy(w3_hbm.at[e_idx], w3buf.at[parity],
                              sems.at[parity]).start()

    @pl.when(jnp.logical_and(new_run, t == 0))
    def _():
        issue(e_t, p)

        @pl.when(has_next)
        def _():
            issue(e_next, lax.rem(run_idx + 1, 3))

    @pl.when(new_run)
    def _():
        pltpu.make_async_copy(w1_hbm.at[0], w1buf.at[p], sems.at[p]).wait()
        pltpu.make_async_copy(w2_hbm.at[0], w2buf.at[p], sems.at[p]).wait()
        pltpu.make_async_copy(w3_hbm.at[0], w3buf.at[p], sems.at[p]).wait()

    @pl.when(jnp.logical_and(new_run, has_next2))
    def _():
        issue(e_next2, lax.rem(run_idx + 2, 3))

    @pl.when(t < total)
    def _():
        h = jnp.maximum(jnp.dot(xs_ref[...], w1buf[p],
                                preferred_element_type=jnp.float32)
                        + b1_ref[0], 0.0)
        h = jnp.maximum(jnp.dot(h, w2buf[p],
                                preferred_element_type=jnp.float32)
                        + b2_ref[0], 0.0)
        o_ref[...] = jnp.dot(h, w3buf[p],
                             preferred_element_type=jnp.float32) + b3_ref[0]


def _expert_mlp(ci, x_sorted, eW1, eb1, eW2, eb2, eW3, eb3):
    def wmap(t, ci_ref):
        return (_expert_of(t, ci_ref), 0, 0)

    grid_spec = pltpu.PrefetchScalarGridSpec(
        num_scalar_prefetch=1,
        grid=(NT,),
        in_specs=[
            pl.BlockSpec((T, D_IN), lambda t, ci_ref: (t, 0)),
            pl.BlockSpec((1, 1, H), wmap),
            pl.BlockSpec((1, 1, H), wmap),
            pl.BlockSpec((1, 1, D_OUT), wmap),
            pl.BlockSpec(memory_space=pl.ANY),
            pl.BlockSpec(memory_space=pl.ANY),
            pl.BlockSpec(memory_space=pl.ANY),
        ],
        out_specs=pl.BlockSpec((T, D_OUT), lambda t, ci_ref: (t, 0)),
        scratch_shapes=[
            pltpu.VMEM((3, D_IN, H), jnp.float32),
            pltpu.VMEM((3, H, H), jnp.float32),
            pltpu.VMEM((3, H, D_OUT), jnp.float32),
            pltpu.SemaphoreType.DMA((3,)),
        ],
    )
    return pl.pallas_call(
        _mlp_body,
        grid_spec=grid_spec,
        out_shape=jax.ShapeDtypeStruct((P, D_OUT), jnp.float32),
        compiler_params=pltpu.CompilerParams(
            dimension_semantics=("arbitrary",)),
    )(ci, x_sorted, eb1.reshape(N_EXPERTS, 1, H),
      eb2.reshape(N_EXPERTS, 1, H), eb3.reshape(N_EXPERTS, 1, D_OUT),
      eW1, eW2, eW3)


def kernel(x, gW1, gb1, gW2, gb2, gW3, gb3, eW1, eb1, eW2, eb2, eW3, eb3):
    logits, cluster_ids, dest, ci = _gating(x, gW1, gb1, gW2, gb2, gW3, gb3)
    x_sorted = _make_row_mover(N_TOKENS, P, D_IN, "dispatch_scatter", True)(x, dest)
    y_sorted = _expert_mlp(ci, x_sorted, eW1, eb1, eW2, eb2, eW3, eb3)
    outputs = _make_row_mover(N_TOKENS, N_TOKENS, D_OUT, "undispatch_gather",
                              False)(y_sorted, dest)
    return outputs, cluster_ids, logits
